# Initial kernel scaffold; baseline (speedup 1.0000x reference)
import jax
import jax.numpy as jnp
from jax.experimental import pallas as pl


def kernel(indices, weight):
    raise NotImplementedError("write your pallas kernel here")


# speedup vs baseline: 1.5591x; 1.5591x over previous
import functools

import jax
import jax.numpy as jnp
from jax import lax
from jax.experimental import pallas as pl
from jax.experimental.pallas import tpu as pltpu
from jax.experimental.pallas import tpu_sc as plsc

NUM_EMB = 1000000
DIM = 32
BATCH = 16384
N_FIELDS = 26
B_TOTAL = BATCH * N_FIELDS

_info = plsc.get_sparse_core_info()
NC = _info.num_cores
NS = _info.num_subcores
NW = NC * NS
B_PER_W = B_TOTAL // NW
CHUNK = 1664
N_CHUNKS = B_PER_W // CHUNK

_mesh = plsc.VectorSubcoreMesh(core_axis_name="c", subcore_axis_name="s")


@functools.partial(
    pl.kernel,
    mesh=_mesh,
    out_type=jax.ShapeDtypeStruct((B_TOTAL, DIM), jnp.float32),
    scratch_types=[
        pltpu.VMEM((CHUNK,), jnp.int32),
        pltpu.VMEM((CHUNK, DIM), jnp.float32),
        pltpu.SemaphoreType.DMA,
    ],
    compiler_params=pltpu.CompilerParams(use_tc_tiling_on_sc=False),
)
def _emb_gather(idx_hbm, table_hbm, out_hbm, idx_v, rows_v, sem):
    wid = lax.axis_index("s") * NC + lax.axis_index("c")
    base = wid * B_PER_W

    def body(i, carry):
        off = base + i * CHUNK
        pltpu.sync_copy(idx_hbm.at[pl.ds(off, CHUNK)], idx_v)
        pltpu.async_copy(table_hbm.at[idx_v], rows_v, sem).wait()
        pltpu.sync_copy(rows_v, out_hbm.at[pl.ds(off, CHUNK)])
        return carry

    lax.fori_loop(0, N_CHUNKS, body, 0)


def kernel(indices, weight):
    flat_idx = indices.reshape(-1).astype(jnp.int32)
    out = _emb_gather(flat_idx, weight)
    return out.reshape(BATCH, N_FIELDS, DIM)


# speedup vs baseline: 1.5756x; 1.0106x over previous
import functools

import jax
import jax.numpy as jnp
from jax import lax
from jax.experimental import pallas as pl
from jax.experimental.pallas import tpu as pltpu
from jax.experimental.pallas import tpu_sc as plsc

NUM_EMB = 1000000
DIM = 32
BATCH = 16384
N_FIELDS = 26
B_TOTAL = BATCH * N_FIELDS

_info = plsc.get_sparse_core_info()
NC = _info.num_cores
NS = _info.num_subcores
NW = NC * NS
B_PER_W = B_TOTAL // NW
CHUNK = 1024
N_CHUNKS = B_PER_W // CHUNK
NBUF = 3

_mesh = plsc.VectorSubcoreMesh(core_axis_name="c", subcore_axis_name="s")


@functools.partial(
    pl.kernel,
    mesh=_mesh,
    out_type=jax.ShapeDtypeStruct((B_TOTAL, DIM), jnp.float32),
    scratch_types=[
        pltpu.VMEM((N_CHUNKS, CHUNK), jnp.int32),
        [pltpu.VMEM((CHUNK, DIM), jnp.float32) for _ in range(NBUF)],
        [pltpu.SemaphoreType.DMA for _ in range(NBUF)],
        [pltpu.SemaphoreType.DMA for _ in range(NBUF)],
    ],
    compiler_params=pltpu.CompilerParams(use_tc_tiling_on_sc=False),
)
def _emb_gather(idx_hbm, table_hbm, out_hbm, idx_v, rows, sem_g, sem_o):
    wid = lax.axis_index("s") * NC + lax.axis_index("c")
    base = wid * B_PER_W

    pltpu.sync_copy(idx_hbm.at[wid], idx_v)

    def start_gather(i, b):
        pltpu.make_async_copy(table_hbm.at[idx_v.at[i]], rows[b], sem_g[b]).start()

    for i in range(NBUF):
        start_gather(i, i)

    for i in range(N_CHUNKS):
        b = i % NBUF
        off = base + i * CHUNK
        pltpu.make_async_copy(table_hbm.at[idx_v.at[i]], rows[b], sem_g[b]).wait()
        store = pltpu.async_copy(rows[b], out_hbm.at[pl.ds(off, CHUNK)], sem_o[b])
        if i + NBUF < N_CHUNKS:
            store.wait()
            start_gather(i + NBUF, b)
        else:
            store.wait()


def kernel(indices, weight):
    flat_idx = indices.reshape(NW, N_CHUNKS, CHUNK).astype(jnp.int32)
    out = _emb_gather(flat_idx, weight)
    return out.reshape(BATCH, N_FIELDS, DIM)


# speedup vs baseline: 1.5761x; 1.0003x over previous
import functools

import jax
import jax.numpy as jnp
from jax import lax
from jax.experimental import pallas as pl
from jax.experimental.pallas import tpu as pltpu
from jax.experimental.pallas import tpu_sc as plsc

NUM_EMB = 1000000
DIM = 32
BATCH = 16384
N_FIELDS = 26
B_TOTAL = BATCH * N_FIELDS

_info = plsc.get_sparse_core_info()
NC = _info.num_cores
NS = _info.num_subcores
NW = NC * NS
ITEMS_PER_W = BATCH // NW
B_PER_W = ITEMS_PER_W * N_FIELDS
CHUNKI = 64
CHUNK = CHUNKI * N_FIELDS
N_CHUNKS = ITEMS_PER_W // CHUNKI
NBUF = 2

_mesh = plsc.VectorSubcoreMesh(core_axis_name="c", subcore_axis_name="s")


@functools.partial(
    pl.kernel,
    mesh=_mesh,
    out_type=jax.ShapeDtypeStruct((BATCH, N_FIELDS, DIM), jnp.float32),
    scratch_types=[
        pltpu.VMEM((B_PER_W,), jnp.int32),
        [pltpu.VMEM((CHUNK, DIM), jnp.float32) for _ in range(NBUF)],
        [pltpu.SemaphoreType.DMA for _ in range(NBUF)],
        [pltpu.SemaphoreType.DMA for _ in range(NBUF)],
    ],
    compiler_params=pltpu.CompilerParams(use_tc_tiling_on_sc=False),
)
def _emb_gather(idx_hbm, table_hbm, out_hbm, idx_v, rows, sem_g, sem_o):
    wid = lax.axis_index("s") * NC + lax.axis_index("c")
    item0 = wid * ITEMS_PER_W

    pltpu.sync_copy(idx_hbm.at[wid], idx_v)

    def gather_copy(i, b):
        idx_c = idx_v.at[pl.ds(i * CHUNK, CHUNK)]
        return pltpu.make_async_copy(table_hbm.at[idx_c], rows[b], sem_g[b])

    def store_chunk(i, b):
        def body(j, carry):
            item = item0 + i * CHUNKI + j
            src = rows[b].at[pl.ds(j * N_FIELDS, N_FIELDS)]
            pltpu.make_async_copy(src, out_hbm.at[item], sem_o[b]).start()
            return carry
        lax.fori_loop(0, CHUNKI, body, 0)

    def drain_chunk(i, b):
        def body(j, carry):
            src = rows[b].at[pl.ds(0, N_FIELDS)]
            pltpu.make_async_copy(src, out_hbm.at[item0], sem_o[b]).wait()
            return carry
        lax.fori_loop(0, CHUNKI, body, 0)

    for i in range(NBUF):
        gather_copy(i, i).start()

    for i in range(N_CHUNKS):
        b = i % NBUF
        gather_copy(i, b).wait()
        store_chunk(i, b)
        drain_chunk(i, b)
        if i + NBUF < N_CHUNKS:
            gather_copy(i + NBUF, b).start()


def kernel(indices, weight):
    flat_idx = indices.reshape(NW, B_PER_W).astype(jnp.int32)
    return _emb_gather(flat_idx, weight)
